# baseline (speedup 1.0000x reference)
import functools

import jax
import jax.numpy as jnp
from jax import lax
from jax.experimental import pallas as pl
from jax.experimental.pallas import tpu as pltpu
from jax.experimental.pallas import tpu_sc as plsc

VOCAB = 1000000
EMBED_DIM = 64
BATCH = 4096
HIST = 200

NUM_CORES = 2
NUM_SUBCORES = 16
NW = NUM_CORES * NUM_SUBCORES

RB = 128
NBLK = (VOCAB // RB)
MAIN_ROWS = NBLK * RB
TAIL = VOCAB - MAIN_ROWS
SR = VOCAB // 2

HTILES = HIST // 8
BBLK = BATCH // 128

_mesh = plsc.VectorSubcoreMesh(
    core_axis_name="c", subcore_axis_name="s",
    num_cores=NUM_CORES, num_subcores=NUM_SUBCORES,
)

_params = pltpu.CompilerParams(
    use_tc_tiling_on_sc=True, needs_layout_passes=False
)


@functools.partial(
    pl.kernel,
    out_type=jax.ShapeDtypeStruct((SR, 128), jnp.float32),
    mesh=_mesh,
    scratch_types=[
        pltpu.VMEM((2, 64, 128), jnp.float32),
        pltpu.VMEM((2, 64, 128), jnp.float32),
        pltpu.VMEM((64, 64), jnp.float32),
        pltpu.SemaphoreType.DMA,
        pltpu.SemaphoreType.DMA,
        pltpu.SemaphoreType.DMA,
        pltpu.SemaphoreType.DMA,
    ],
    compiler_params=_params,
)
def _convert(tt_hbm, tail_hbm, scr_hbm, tile_v, pair_v, tail_v,
             gi0, gi1, wo0, wo1):
    wid = lax.axis_index("s") * NUM_CORES + lax.axis_index("c")
    gsem = (gi0, gi1)
    wsem = (wo0, wo1)

    e_lo = jnp.arange(16, dtype=jnp.int32)
    e_vecs = [e_lo + 16 * g for g in range(4)]

    def tile_in_start(k, buf):
        pltpu.make_async_copy(
            tt_hbm.at[:, pl.ds(k * RB, RB)], tile_v.at[buf], gsem[buf]
        ).start()

    def tile_in_wait(buf):
        pltpu.make_async_copy(
            tt_hbm.at[:, pl.ds(0, RB)], tile_v.at[buf], gsem[buf]
        ).wait()

    def pair_out_start(k, buf):
        pltpu.make_async_copy(
            pair_v.at[buf], scr_hbm.at[pl.ds(k * (RB // 2), RB // 2)],
            wsem[buf],
        ).start()

    def pair_out_wait(buf):
        pltpu.make_async_copy(
            pair_v.at[buf], scr_hbm.at[pl.ds(0, RB // 2)], wsem[buf]
        ).wait()

    def transpose_block(buf):
        @pl.loop(0, 64, unroll=8)
        def _(r_loc):
            for half in range(2):
                src = tile_v.at[buf]
                r_rel = jnp.full((16,), 2 * r_loc + half, jnp.int32)
                for g in range(4):
                    vals = plsc.load_gather(src, [e_vecs[g], r_rel])
                    pair_v[buf, r_loc, pl.ds(half * 64 + g * 16, 16)] = vals

    nmine = (NBLK - wid + NW - 1) // NW

    tile_in_start(wid, 0)

    @pl.loop(0, nmine, step=2)
    def _(i):
        k = wid + i * NW

        @pl.when(i + 1 < nmine)
        def _():
            tile_in_start(k + NW, 1)
        tile_in_wait(0)

        @pl.when(i >= 2)
        def _():
            pair_out_wait(0)
        transpose_block(0)
        pair_out_start(k, 0)

        @pl.when(i + 1 < nmine)
        def _():
            @pl.when(i + 2 < nmine)
            def _():
                tile_in_start(k + 2 * NW, 0)
            tile_in_wait(1)

            @pl.when(i >= 1)
            def _():
                pair_out_wait(1)
            transpose_block(1)
            pair_out_start(k + NW, 1)

    pair_out_wait(0)
    pair_out_wait(1)

    @pl.when(wid == 0)
    def _():
        pltpu.sync_copy(tail_hbm, tail_v)

        @pl.loop(0, 32, unroll=8)
        def _(r_loc):
            for half in range(2):
                r_rel = jnp.full((16,), 2 * r_loc + half, jnp.int32)
                for g in range(4):
                    vals = plsc.load_gather(tail_v, [r_rel, e_vecs[g]])
                    pair_v[0, r_loc, pl.ds(half * 64 + g * 16, 16)] = vals
        pltpu.sync_copy(
            pair_v.at[0, pl.ds(0, 32)], scr_hbm.at[pl.ds(SR - 32, 32)]
        )


@functools.partial(
    pl.kernel,
    out_type=jax.ShapeDtypeStruct((HIST, EMBED_DIM, BATCH), jnp.float32),
    mesh=_mesh,
    scratch_types=[
        pltpu.VMEM((8, 128), jnp.int32),
        pltpu.VMEM((8, 128), jnp.int32),
        pltpu.VMEM((8, 128), jnp.int32),
        pltpu.VMEM((2, 128, 128), jnp.float32),
        pltpu.VMEM((2, 64, 128), jnp.float32),
        pltpu.SemaphoreType.DMA,
        pltpu.SemaphoreType.DMA,
        pltpu.SemaphoreType.DMA,
        pltpu.SemaphoreType.DMA,
    ],
    compiler_params=_params,
)
def _lookup(xt_hbm, scr_hbm, out_hbm, idx_v, rid_v, par_v, rows_v, outt_v,
            g0, g1, w0, w1):
    wid = lax.axis_index("s") * NUM_CORES + lax.axis_index("c")
    b0 = wid * 128
    gsem = (g0, g1)
    wsem = (w0, w1)

    lanes = jnp.arange(16, dtype=jnp.int32)
    j_vecs = [lanes + 16 * g for g in range(8)]

    def gather_start(t, buf):
        pltpu.make_async_copy(
            scr_hbm.at[rid_v.at[t]], rows_v.at[buf], gsem[buf]
        ).start()

    def gather_wait(buf):
        pltpu.make_async_copy(
            scr_hbm.at[rid_v.at[0]], rows_v.at[buf], gsem[buf]
        ).wait()

    def out_start(h, buf):
        pltpu.make_async_copy(
            outt_v.at[buf], out_hbm.at[h, :, pl.ds(b0, 128)], wsem[buf]
        ).start()

    def out_wait(buf):
        pltpu.make_async_copy(
            outt_v.at[buf], out_hbm.at[0, :, pl.ds(b0, 128)], wsem[buf]
        ).wait()

    @pl.loop(0, HTILES)
    def _(hg):
        pltpu.sync_copy(xt_hbm.at[pl.ds(hg * 8, 8), pl.ds(b0, 128)], idx_v)
        for t in range(8):
            for g in range(8):
                v = idx_v[t, pl.ds(g * 16, 16)]
                rid_v[t, pl.ds(g * 16, 16)] = lax.shift_right_logical(v, 1)
                par_v[t, pl.ds(g * 16, 16)] = (v & 1) * 64

        gather_start(0, 0)
        for t in range(8):
            buf = t % 2
            if t + 1 < 8:
                if t >= 1:
                    out_wait(1 - buf)
                gather_start(t + 1, 1 - buf)
            gather_wait(buf)
            pbase = [par_v[t, pl.ds(g * 16, 16)] for g in range(8)]
            src = rows_v.at[buf]

            @pl.loop(0, EMBED_DIM, unroll=4)
            def _(e):
                for g in range(8):
                    vals = plsc.load_gather(src, [j_vecs[g], pbase[g] + e])
                    outt_v[buf, e, pl.ds(g * 16, 16)] = vals
            out_start(hg * 8 + t, buf)
        out_wait(0)
        out_wait(1)


def kernel(x, table):
    tt = table.T
    tail = table[MAIN_ROWS:, :]
    scr = _convert(tt, tail)
    xt = x.T
    out = _lookup(xt, scr)
    return out.transpose(2, 0, 1)


# speedup vs baseline: 1.1782x; 1.1782x over previous
import functools

import jax
import jax.numpy as jnp
from jax import lax
from jax.experimental import pallas as pl
from jax.experimental.pallas import tpu as pltpu
from jax.experimental.pallas import tpu_sc as plsc

VOCAB = 1000000
EMBED_DIM = 64
BATCH = 4096
HIST = 200

NUM_CORES = 2
NUM_SUBCORES = 16
NW = NUM_CORES * NUM_SUBCORES

RB = 128
NBLK = (VOCAB // RB)
MAIN_ROWS = NBLK * RB
TAIL = VOCAB - MAIN_ROWS
SR = VOCAB // 2

HTILES = HIST // 8
BBLK = BATCH // 128

_mesh = plsc.VectorSubcoreMesh(
    core_axis_name="c", subcore_axis_name="s",
    num_cores=NUM_CORES, num_subcores=NUM_SUBCORES,
)

_params = pltpu.CompilerParams(
    use_tc_tiling_on_sc=True, needs_layout_passes=False
)


@functools.partial(
    pl.kernel,
    out_type=jax.ShapeDtypeStruct((SR, 128), jnp.float32),
    mesh=_mesh,
    scratch_types=[
        pltpu.VMEM((2, 64, 128), jnp.float32),
        pltpu.VMEM((64, 130), jnp.float32),
        pltpu.VMEM((2, 64, 128), jnp.float32),
        pltpu.VMEM((64, 64), jnp.float32),
        pltpu.SemaphoreType.DMA,
        pltpu.SemaphoreType.DMA,
        pltpu.SemaphoreType.DMA,
        pltpu.SemaphoreType.DMA,
    ],
    compiler_params=_params,
)
def _convert(tt_hbm, tail_hbm, scr_hbm, tile_v, pad_v, pair_v, tail_v,
             gi0, gi1, wo0, wo1):
    wid = lax.axis_index("s") * NUM_CORES + lax.axis_index("c")
    gsem = (gi0, gi1)
    wsem = (wo0, wo1)

    lanes = jnp.arange(16, dtype=jnp.int32)
    rg_row = [(rg * 16 + lanes) // 2 for rg in range(8)]
    rg_col = [((rg * 16 + lanes) & 1) * 65 for rg in range(8)]
    e_lo = jnp.arange(16, dtype=jnp.int32)
    e_vecs = [e_lo + 16 * g for g in range(4)]

    def tile_in_start(k, buf):
        pltpu.make_async_copy(
            tt_hbm.at[:, pl.ds(k * RB, RB)], tile_v.at[buf], gsem[buf]
        ).start()

    def tile_in_wait(buf):
        pltpu.make_async_copy(
            tt_hbm.at[:, pl.ds(0, RB)], tile_v.at[buf], gsem[buf]
        ).wait()

    def pair_out_start(k, buf):
        pltpu.make_async_copy(
            pair_v.at[buf], scr_hbm.at[pl.ds(k * (RB // 2), RB // 2)],
            wsem[buf],
        ).start()

    def pair_out_wait(buf):
        pltpu.make_async_copy(
            pair_v.at[buf], scr_hbm.at[pl.ds(0, RB // 2)], wsem[buf]
        ).wait()

    def transpose_block(buf):
        @pl.loop(0, 64, unroll=4)
        def _(e):
            src = tile_v.at[buf, e]
            e_splat = jnp.full((16,), e, jnp.int32)
            for rg in range(8):
                vals = src[pl.ds(rg * 16, 16)]
                plsc.store_scatter(
                    pad_v, [rg_row[rg], rg_col[rg] + e_splat], vals)

        @pl.loop(0, 64, unroll=4)
        def _(r_loc):
            for half in range(2):
                for g in range(4):
                    vals = pad_v[r_loc, pl.ds(half * 65 + g * 16, 16)]
                    pair_v[buf, r_loc, pl.ds(half * 64 + g * 16, 16)] = vals

    nmine = (NBLK - wid + NW - 1) // NW

    tile_in_start(wid, 0)

    @pl.loop(0, nmine, step=2)
    def _(i):
        k = wid + i * NW

        @pl.when(i + 1 < nmine)
        def _():
            tile_in_start(k + NW, 1)
        tile_in_wait(0)

        @pl.when(i >= 2)
        def _():
            pair_out_wait(0)
        transpose_block(0)
        pair_out_start(k, 0)

        @pl.when(i + 1 < nmine)
        def _():
            @pl.when(i + 2 < nmine)
            def _():
                tile_in_start(k + 2 * NW, 0)
            tile_in_wait(1)

            @pl.when(i >= 1)
            def _():
                pair_out_wait(1)
            transpose_block(1)
            pair_out_start(k + NW, 1)

    pair_out_wait(0)
    pair_out_wait(1)

    @pl.when(wid == 0)
    def _():
        pltpu.sync_copy(tail_hbm, tail_v)

        @pl.loop(0, 32, unroll=8)
        def _(r_loc):
            for half in range(2):
                r_rel = jnp.full((16,), 2 * r_loc + half, jnp.int32)
                for g in range(4):
                    vals = plsc.load_gather(tail_v, [r_rel, e_vecs[g]])
                    pair_v[0, r_loc, pl.ds(half * 64 + g * 16, 16)] = vals
        pltpu.sync_copy(
            pair_v.at[0, pl.ds(0, 32)], scr_hbm.at[pl.ds(SR - 32, 32)]
        )


@functools.partial(
    pl.kernel,
    out_type=jax.ShapeDtypeStruct((HIST, EMBED_DIM, BATCH), jnp.float32),
    mesh=_mesh,
    scratch_types=[
        pltpu.VMEM((8, 128), jnp.int32),
        pltpu.VMEM((8, 128), jnp.int32),
        pltpu.VMEM((8, 128), jnp.int32),
        pltpu.VMEM((2, 128, 129), jnp.float32),
        pltpu.VMEM((2, 64, 130), jnp.float32),
        pltpu.SemaphoreType.DMA,
        pltpu.SemaphoreType.DMA,
        pltpu.SemaphoreType.DMA,
        pltpu.SemaphoreType.DMA,
    ],
    compiler_params=_params,
)
def _lookup(xt_hbm, scr_hbm, out_hbm, idx_v, rid_v, par_v, rows_v, outt_v,
            g0, g1, w0, w1):
    wid = lax.axis_index("s") * NUM_CORES + lax.axis_index("c")
    b0 = wid * 128
    gsem = (g0, g1)
    wsem = (w0, w1)

    lanes = jnp.arange(16, dtype=jnp.int32)
    j_vecs = [lanes + 16 * g for g in range(8)]

    def gather_start(t, buf):
        pltpu.make_async_copy(
            scr_hbm.at[rid_v.at[t]], rows_v.at[buf, :, pl.ds(0, 128)],
            gsem[buf],
        ).start()

    def gather_wait(buf):
        pltpu.make_async_copy(
            scr_hbm.at[rid_v.at[0]], rows_v.at[buf, :, pl.ds(0, 128)],
            gsem[buf],
        ).wait()

    def out_start(h, buf):
        pltpu.make_async_copy(
            outt_v.at[buf, :, pl.ds(0, 128)], out_hbm.at[h, :, pl.ds(b0, 128)],
            wsem[buf],
        ).start()

    def out_wait(buf):
        pltpu.make_async_copy(
            outt_v.at[buf, :, pl.ds(0, 128)], out_hbm.at[0, :, pl.ds(b0, 128)],
            wsem[buf],
        ).wait()

    @pl.loop(0, HTILES)
    def _(hg):
        pltpu.sync_copy(xt_hbm.at[pl.ds(hg * 8, 8), pl.ds(b0, 128)], idx_v)
        for t in range(8):
            for g in range(8):
                v = idx_v[t, pl.ds(g * 16, 16)]
                rid_v[t, pl.ds(g * 16, 16)] = lax.shift_right_logical(v, 1)
                par_v[t, pl.ds(g * 16, 16)] = (v & 1) * 64

        gather_start(0, 0)
        for t in range(8):
            buf = t % 2
            if t + 1 < 8:
                if t >= 1:
                    out_wait(1 - buf)
                gather_start(t + 1, 1 - buf)
            gather_wait(buf)
            src = rows_v.at[buf]
            pcol = [par_v[t, pl.ds(g * 16, 16)] for g in range(8)]

            @pl.loop(0, EMBED_DIM, unroll=4)
            def _(e):
                for g in range(8):
                    vals = plsc.load_gather(src, [j_vecs[g], pcol[g] + e])
                    outt_v[buf, e, pl.ds(g * 16, 16)] = vals
            out_start(hg * 8 + t, buf)
        out_wait(0)
        out_wait(1)


def kernel(x, table):
    tt = table.T
    tail = table[MAIN_ROWS:, :]
    scr = _convert(tt, tail)
    xt = x.T
    out = _lookup(xt, scr)
    return out.transpose(2, 0, 1)


# speedup vs baseline: 2.2679x; 1.9249x over previous
import functools

import jax
import jax.numpy as jnp
from jax import lax
from jax.experimental import pallas as pl
from jax.experimental.pallas import tpu as pltpu
from jax.experimental.pallas import tpu_sc as plsc

VOCAB = 1000000
EMBED_DIM = 64
BATCH = 4096
HIST = 200

NUM_CORES = 2
NUM_SUBCORES = 16

B_TOTAL = BATCH * HIST
NW = NUM_CORES * NUM_SUBCORES
B_PER_W = B_TOTAL // NW
CHUNK = 128
NCHUNK = B_PER_W // CHUNK
NBUF = 2

_mesh = plsc.VectorSubcoreMesh(
    core_axis_name="c", subcore_axis_name="s",
    num_cores=NUM_CORES, num_subcores=NUM_SUBCORES,
)


@functools.partial(
    pl.kernel,
    out_type=jax.ShapeDtypeStruct((B_TOTAL, EMBED_DIM), jnp.float32),
    mesh=_mesh,
    scratch_types=[
        pltpu.VMEM((NCHUNK, CHUNK), jnp.int32),
        pltpu.VMEM((NBUF, CHUNK, EMBED_DIM), jnp.float32),
        pltpu.SemaphoreType.DMA,
        pltpu.SemaphoreType.DMA,
        pltpu.SemaphoreType.DMA,
        pltpu.SemaphoreType.DMA,
    ],
    compiler_params=pltpu.CompilerParams(use_tc_tiling_on_sc=False),
)
def _sc_gather(x_hbm, table_hbm, out_hbm, idx_v, rows_v, g0, g1, w0, w1):
    wid = lax.axis_index("s") * NUM_CORES + lax.axis_index("c")
    base = wid * B_PER_W

    gsem = (g0, g1)
    wsem = (w0, w1)

    pltpu.sync_copy(x_hbm.at[pl.ds(wid * NCHUNK, NCHUNK)], idx_v)

    def gather_start(j, buf):
        pltpu.make_async_copy(
            table_hbm.at[idx_v.at[j]], rows_v.at[buf], gsem[buf]
        ).start()

    def gather_wait(buf):
        pltpu.make_async_copy(
            table_hbm.at[idx_v.at[0]], rows_v.at[buf], gsem[buf]
        ).wait()

    def write_start(j, buf):
        pltpu.make_async_copy(
            rows_v.at[buf], out_hbm.at[pl.ds(base + j * CHUNK, CHUNK)],
            wsem[buf],
        ).start()

    def write_wait(buf):
        pltpu.make_async_copy(
            rows_v.at[buf], out_hbm.at[pl.ds(base, CHUNK)], wsem[buf]
        ).wait()

    gather_start(0, 0)
    gather_start(1, 1)
    gather_wait(0)
    write_start(0, 0)

    @pl.loop(1, NCHUNK - 1, step=NBUF)
    def _(j0):
        for b in range(NBUF):
            j = j0 + b
            buf = (1 + b) % NBUF
            nxt = (b) % NBUF
            write_wait(nxt)
            gather_start(j + 1, nxt)
            gather_wait(buf)
            write_start(j, buf)

    gather_wait((NCHUNK - 1) % NBUF)
    write_start(NCHUNK - 1, (NCHUNK - 1) % NBUF)
    write_wait((NCHUNK - 2) % NBUF)
    write_wait((NCHUNK - 1) % NBUF)


def kernel(x, table):
    x2 = x.reshape(NW * NCHUNK, CHUNK)
    out = _sc_gather(x2, table)
    return out.reshape(BATCH, HIST, EMBED_DIM)
